# Initial kernel scaffold; baseline (speedup 1.0000x reference)
import jax
import jax.numpy as jnp
from jax.experimental import pallas as pl


def kernel(pos_x, pos_edge_index, neg_x, neg_edge_index, pos_W0, pos_W1, pos_W2, neg_W0, neg_W1, neg_W2):
    raise NotImplementedError("write your pallas kernel here")


# speedup vs baseline: 2.2114x; 2.2114x over previous
import functools

import jax
import jax.numpy as jnp
from jax import lax
from jax.experimental import pallas as pl
from jax.experimental.pallas import tpu as pltpu
from jax.experimental.pallas import tpu_sc as plsc

N_NODES = 10000
N_EDGES = 320000
F = 128

NC = 2
NS = 16
NW = NC * NS
K = 128
EDGES_PER_TILE = N_EDGES // NW
NCHUNK = 80
PAD_EDGES = NW * NCHUNK * K
ACC_ROWS = 10112
ZR = ACC_ROWS // NS
WA = 624
WTAIL = N_NODES - NS * WA

_sc_mesh = plsc.VectorSubcoreMesh(
    core_axis_name="c", subcore_axis_name="s", num_cores=NC, num_subcores=NS)


def _sc_body(g_hbm, src_hbm, dst_hbm, zero_hbm, out_hbm,
             src_v, dst_v, rows_v, acc, sem):
    c = lax.axis_index("c")
    s = lax.axis_index("s")
    w = c * NS + s
    pltpu.sync_copy(zero_hbm.at[pl.ds(s * ZR, ZR)], acc.at[pl.ds(s * ZR, ZR)])
    pltpu.sync_copy(src_hbm.at[w], src_v)
    pltpu.sync_copy(dst_hbm.at[w], dst_v)
    plsc.subcore_barrier()

    def chunk(j, carry):
        pltpu.async_copy(g_hbm.at[src_v.at[j]], rows_v, sem).wait()
        pltpu.sync_copy(rows_v, acc.at[dst_v.at[j]], add=True)
        return carry

    lax.fori_loop(0, NCHUNK, chunk, 0)
    plsc.subcore_barrier()
    pltpu.sync_copy(acc.at[pl.ds(s * WA, WA)], out_hbm.at[c, pl.ds(s * WA, WA)])

    @pl.when(s == NS - 1)
    def _write_tail():
        pltpu.sync_copy(acc.at[pl.ds(NS * WA, WTAIL)],
                        out_hbm.at[c, pl.ds(NS * WA, WTAIL)])


_sc_scatter = functools.partial(
    pl.kernel,
    out_type=jax.ShapeDtypeStruct((NC, N_NODES, F), jnp.float32),
    mesh=_sc_mesh,
    scratch_types=[
        pltpu.VMEM((NCHUNK, K), jnp.int32),
        pltpu.VMEM((NCHUNK, K), jnp.int32),
        pltpu.VMEM((K, F), jnp.float32),
        pltpu.VMEM_SHARED((ACC_ROWS, F), jnp.float32),
        pltpu.SemaphoreType.DMA,
    ],
)(_sc_body)


ROWS_B = 1000


def _mm0_body(x_ref, w_ref, o_ref):
    o_ref[...] = jnp.dot(x_ref[...], w_ref[...],
                         preferred_element_type=jnp.float32)


def _mid_body(p_ref, w_ref, o_ref):
    a = jnp.maximum(p_ref[0] + p_ref[1], 0.0)
    o_ref[...] = jnp.dot(a, w_ref[...], preferred_element_type=jnp.float32)


def _last_body(p_ref, o_ref):
    o_ref[...] = jnp.maximum(p_ref[0] + p_ref[1], 0.0)


_GRID = N_NODES // ROWS_B
_x_spec = pl.BlockSpec((ROWS_B, F), lambda i: (i, 0))
_w_spec = pl.BlockSpec((F, F), lambda i: (0, 0))
_p_spec = pl.BlockSpec((NC, ROWS_B, F), lambda i: (0, i, 0))
_o_spec = pl.BlockSpec((ROWS_B, F), lambda i: (i, 0))
_o_type = jax.ShapeDtypeStruct((N_NODES, F), jnp.float32)

_mm0 = pl.pallas_call(_mm0_body, grid=(_GRID,), in_specs=[_x_spec, _w_spec],
                      out_specs=_o_spec, out_shape=_o_type)
_mid = pl.pallas_call(_mid_body, grid=(_GRID,), in_specs=[_p_spec, _w_spec],
                      out_specs=_o_spec, out_shape=_o_type)
_last = pl.pallas_call(_last_body, grid=(_GRID,), in_specs=[_p_spec],
                       out_specs=_o_spec, out_shape=_o_type)


def _prep_edges(edge_index):
    pad = PAD_EDGES - N_EDGES
    src = jnp.concatenate(
        [edge_index[0], jnp.zeros((pad,), jnp.int32)]).reshape(NW, NCHUNK, K)
    dst = jnp.concatenate(
        [edge_index[1], jnp.full((pad,), N_NODES, jnp.int32)]).reshape(
            NW, NCHUNK, K)
    return src, dst


def _branch(x, edge_index, W0, W1, W2, zeros_hbm):
    src, dst = _prep_edges(edge_index)
    g = _mm0(x, W0)
    p = _sc_scatter(g, src, dst, zeros_hbm)
    g = _mid(p, W1)
    p = _sc_scatter(g, src, dst, zeros_hbm)
    g = _mid(p, W2)
    p = _sc_scatter(g, src, dst, zeros_hbm)
    h = _last(p)
    return jnp.concatenate([x, h], axis=1)


def kernel(pos_x, pos_edge_index, neg_x, neg_edge_index,
           pos_W0, pos_W1, pos_W2, neg_W0, neg_W1, neg_W2):
    zeros_hbm = jnp.zeros((ACC_ROWS, F), jnp.float32)
    pos = _branch(pos_x, pos_edge_index, pos_W0, pos_W1, pos_W2, zeros_hbm)
    neg = _branch(neg_x, neg_edge_index, neg_W0, neg_W1, neg_W2, zeros_hbm)
    return jnp.concatenate([pos, neg], axis=1)


# speedup vs baseline: 3.4999x; 1.5827x over previous
import functools

import jax
import jax.numpy as jnp
from jax import lax
from jax.experimental import pallas as pl
from jax.experimental.pallas import tpu as pltpu
from jax.experimental.pallas import tpu_sc as plsc

N_NODES = 10000
N_EDGES = 320000
F = 128
FH = F // 2

NC = 2
NS = 16
K = 128
NCHUNK = 160
PAD_EDGES = NS * NCHUNK * K
ACC_ROWS = 10112
ZR = ACC_ROWS // NS
WA = 624
WTAIL = N_NODES - NS * WA
NBUF = 5
NGROUP = NCHUNK // NBUF

_sc_mesh = plsc.VectorSubcoreMesh(
    core_axis_name="c", subcore_axis_name="s", num_cores=NC, num_subcores=NS)


def _sc_body(g_hbm, src_hbm, dst_hbm, zero_hbm, out_hbm,
             src_v, dst_v, rows_v, sem_g, sem_s, acc):
    c = lax.axis_index("c")
    s = lax.axis_index("s")
    gh = g_hbm.at[c]
    pltpu.sync_copy(zero_hbm.at[pl.ds(s * ZR, ZR)], acc.at[pl.ds(s * ZR, ZR)])
    pltpu.sync_copy(src_hbm.at[s], src_v)
    pltpu.sync_copy(dst_hbm.at[s], dst_v)
    plsc.subcore_barrier()

    for b in range(NBUF):
        pltpu.async_copy(gh.at[src_v.at[b]], rows_v.at[b], sem_g.at[b])

    def group(g, carry):
        j0 = g * NBUF
        for b in range(NBUF):
            pltpu.make_async_copy(
                gh.at[src_v.at[j0 + b]], rows_v.at[b], sem_g.at[b]).wait()
            pltpu.async_copy(
                rows_v.at[b], acc.at[dst_v.at[j0 + b]], sem_s.at[b], add=True)
        for b in range(NBUF):
            pltpu.make_async_copy(
                rows_v.at[b], acc.at[dst_v.at[j0 + b]], sem_s.at[b]).wait()
            jn = jnp.minimum(j0 + NBUF + b, NCHUNK - 1)

            @pl.when(g < NGROUP - 1)
            def _next_gather():
                pltpu.async_copy(gh.at[src_v.at[jn]], rows_v.at[b],
                                 sem_g.at[b])

        return carry

    lax.fori_loop(0, NGROUP, group, 0)
    plsc.subcore_barrier()
    pltpu.sync_copy(acc.at[pl.ds(s * WA, WA)], out_hbm.at[c, pl.ds(s * WA, WA)])

    @pl.when(s == NS - 1)
    def _write_tail():
        pltpu.sync_copy(acc.at[pl.ds(NS * WA, WTAIL)],
                        out_hbm.at[c, pl.ds(NS * WA, WTAIL)])


_sc_scatter = functools.partial(
    pl.kernel,
    out_type=jax.ShapeDtypeStruct((NC, N_NODES, FH), jnp.float32),
    mesh=_sc_mesh,
    scratch_types=[
        pltpu.VMEM((NCHUNK, K), jnp.int32),
        pltpu.VMEM((NCHUNK, K), jnp.int32),
        pltpu.VMEM((NBUF, K, FH), jnp.float32),
        pltpu.SemaphoreType.DMA((NBUF,)),
        pltpu.SemaphoreType.DMA((NBUF,)),
        pltpu.VMEM_SHARED((ACC_ROWS, FH), jnp.float32),
    ],
    compiler_params=pltpu.CompilerParams(use_tc_tiling_on_sc=False),
)(_sc_body)


ROWS_B = 1000


def _mm0_body(x_ref, w_ref, o_ref):
    g = jnp.dot(x_ref[...], w_ref[...], preferred_element_type=jnp.float32)
    o_ref[0] = g[:, :FH]
    o_ref[1] = g[:, FH:]


def _mid_body(p_ref, w_ref, o_ref):
    a = jnp.maximum(jnp.concatenate([p_ref[0], p_ref[1]], axis=1), 0.0)
    g = jnp.dot(a, w_ref[...], preferred_element_type=jnp.float32)
    o_ref[0] = g[:, :FH]
    o_ref[1] = g[:, FH:]


def _last_body(p_ref, o_ref):
    o_ref[...] = jnp.maximum(
        jnp.concatenate([p_ref[0], p_ref[1]], axis=1), 0.0)


_GRID = N_NODES // ROWS_B
_x_spec = pl.BlockSpec((ROWS_B, F), lambda i: (i, 0))
_w_spec = pl.BlockSpec((F, F), lambda i: (0, 0))
_p_spec = pl.BlockSpec((NC, ROWS_B, FH), lambda i: (0, i, 0))
_g_spec = pl.BlockSpec((NC, ROWS_B, FH), lambda i: (0, i, 0))
_g_type = jax.ShapeDtypeStruct((NC, N_NODES, FH), jnp.float32)
_h_spec = pl.BlockSpec((ROWS_B, F), lambda i: (i, 0))
_h_type = jax.ShapeDtypeStruct((N_NODES, F), jnp.float32)

_mm0 = pl.pallas_call(_mm0_body, grid=(_GRID,), in_specs=[_x_spec, _w_spec],
                      out_specs=_g_spec, out_shape=_g_type)
_mid = pl.pallas_call(_mid_body, grid=(_GRID,), in_specs=[_p_spec, _w_spec],
                      out_specs=_g_spec, out_shape=_g_type)
_last = pl.pallas_call(_last_body, grid=(_GRID,), in_specs=[_p_spec],
                       out_specs=_h_spec, out_shape=_h_type)


def _prep_edges(edge_index):
    pad = PAD_EDGES - N_EDGES
    src = jnp.concatenate(
        [edge_index[0], jnp.zeros((pad,), jnp.int32)]).reshape(NS, NCHUNK, K)
    dst = jnp.concatenate(
        [edge_index[1], jnp.full((pad,), N_NODES, jnp.int32)]).reshape(
            NS, NCHUNK, K)
    return src, dst


def _branch(x, edge_index, W0, W1, W2, zeros_hbm):
    src, dst = _prep_edges(edge_index)
    g = _mm0(x, W0)
    p = _sc_scatter(g, src, dst, zeros_hbm)
    g = _mid(p, W1)
    p = _sc_scatter(g, src, dst, zeros_hbm)
    g = _mid(p, W2)
    p = _sc_scatter(g, src, dst, zeros_hbm)
    h = _last(p)
    return jnp.concatenate([x, h], axis=1)


def kernel(pos_x, pos_edge_index, neg_x, neg_edge_index,
           pos_W0, pos_W1, pos_W2, neg_W0, neg_W1, neg_W2):
    zeros_hbm = jnp.zeros((ACC_ROWS, FH), jnp.float32)
    pos = _branch(pos_x, pos_edge_index, pos_W0, pos_W1, pos_W2, zeros_hbm)
    neg = _branch(neg_x, neg_edge_index, neg_W0, neg_W1, neg_W2, zeros_hbm)
    return jnp.concatenate([pos, neg], axis=1)
